# Initial kernel scaffold; baseline (speedup 1.0000x reference)
import jax
import jax.numpy as jnp
from jax.experimental import pallas as pl


def kernel(feats, y_hat, y, point_queue, cluster_center, point_queue_ptr):
    raise NotImplementedError("write your pallas kernel here")


# speedup vs baseline: 1.0016x; 1.0016x over previous
import numpy as np
import jax
import jax.numpy as jnp
from jax.experimental import pallas as pl
from jax.experimental.pallas import tpu as pltpu

C = 17
DIM = 64
NVIEW = 100
QSIZE = 150
UPD = 30
MU = 0.99
TEMP = 0.1
BTEMP = 1.0

NCON = C * QSIZE + C
NCON_PAD = 2688

_tie_cache = {}


def _ties(B, N):
    k = (B, N)
    if k not in _tie_cache:
        with jax.ensure_compile_time_eval():
            tie = np.asarray(jax.random.uniform(jax.random.key(42), (B, C, N),
                                                dtype=jnp.float32))
            tie2 = np.asarray(jax.random.uniform(jax.random.key(7), (C, B * NVIEW),
                                                 dtype=jnp.float32))
        _tie_cache[k] = (tie, tie2)
    return _tie_cache[k]


def _normalize(x, axis=-1, eps=1e-12):
    return x / jnp.maximum(jnp.linalg.norm(x, axis=axis, keepdims=True), eps)


RB = 400


def _loss_body(anchor_ref, ct_ref, valid_ref, out_ref):
    i = pl.program_id(0)
    a = anchor_ref[...]
    logits = jax.lax.dot_general(
        a, ct_ref[...], (((1,), (0,)), ((), ())),
        preferred_element_type=jnp.float32,
        precision=jax.lax.Precision.HIGHEST) * (1.0 / TEMP)
    col = jax.lax.broadcasted_iota(jnp.int32, (RB, NCON_PAD), 1)
    col_ok = col < NCON
    clab = jnp.where(col < C * QSIZE, col // QSIZE, col - C * QSIZE)
    row = jax.lax.broadcasted_iota(jnp.int32, (RB, NCON_PAD), 0)
    alab = ((i * RB + row) // NVIEW) % C
    pos = (alab == clab) & col_ok
    lm = jnp.where(col_ok, logits, jnp.float32(-1e30))
    m = jnp.max(lm, axis=1, keepdims=True)
    sh = lm - m
    ex = jnp.where(col_ok, jnp.exp(sh), 0.0)
    lse = jnp.log(jnp.sum(ex, axis=1, keepdims=True))
    lp = sh - lse
    s_pos = jnp.sum(jnp.where(pos, lp, 0.0), axis=1)
    per = (-(TEMP / BTEMP) / (QSIZE + 1.0)) * s_pos
    blk = jnp.sum(per * valid_ref[0, 0, :])

    @pl.when(i == 0)
    def _():
        out_ref[...] = jnp.zeros((1, 1), jnp.float32)

    out_ref[...] += blk[None, None]


def _loss_pallas(anchor, contrastT, valid):
    na = anchor.shape[0]
    nblk = na // RB
    valid3 = valid.reshape(nblk, 1, RB)
    out = pl.pallas_call(
        _loss_body,
        grid=(nblk,),
        in_specs=[
            pl.BlockSpec((RB, DIM), lambda i: (i, 0)),
            pl.BlockSpec((DIM, NCON_PAD), lambda i: (0, 0)),
            pl.BlockSpec((1, 1, RB), lambda i: (i, 0, 0)),
        ],
        out_specs=pl.BlockSpec((1, 1), lambda i: (0, 0)),
        out_shape=jax.ShapeDtypeStruct((1, 1), jnp.float32),
    )(anchor, contrastT, valid3)
    return out[0, 0]


def kernel(feats, y_hat, y, point_queue, cluster_center, point_queue_ptr):
    B, N, _ = feats.shape
    tie, tie2 = _ties(B, N)
    cls = jnp.arange(C)

    mask = (y_hat[:, None, :] == cls[None, :, None])
    hard = mask & (y[:, None, :] != cls[None, :, None])
    score = mask.astype(jnp.float32) * 4.0 + hard.astype(jnp.float32) + jnp.asarray(tie)
    _, idx = jax.lax.top_k(score, NVIEW)
    b_idx = jnp.arange(B)[:, None, None]
    X_sel = feats[b_idx, idx]
    valid = (y_hat[b_idx, idx] == cls[None, :, None])

    anchor = _normalize(X_sel.reshape(-1, DIM))
    anchor_valid = valid.reshape(-1).astype(jnp.float32)

    q_flat = point_queue.reshape(C * QSIZE, DIM)
    contrast = jnp.concatenate([q_flat, cluster_center], axis=0)
    contrastT = jnp.zeros((DIM, NCON_PAD), jnp.float32).at[:, :NCON].set(contrast.T)

    wloss = _loss_pallas(anchor, contrastT, anchor_valid)
    loss = wloss / jnp.maximum(jnp.sum(anchor_valid), 1.0)

    anch_c = jnp.transpose(X_sel, (1, 0, 2, 3)).reshape(C, B * NVIEW, DIM)
    val_c = jnp.transpose(valid, (1, 0, 2)).reshape(C, B * NVIEW).astype(jnp.float32)
    _, idx2 = jax.lax.top_k(val_c * 2.0 + jnp.asarray(tie2), UPD)
    upd = jnp.take_along_axis(anch_c, idx2[:, :, None], axis=1)
    upd = _normalize(upd, axis=2)
    ptr = point_queue_ptr.astype(jnp.int32)
    pos_idx = (ptr[:, None] + jnp.arange(UPD)[None, :]) % QSIZE
    c_rows = jnp.broadcast_to(cls[:, None], (C, UPD))
    new_queue = point_queue.at[c_rows, pos_idx].set(upd)
    new_ptr = (ptr + UPD) % QSIZE

    anch_n = _normalize(anch_c, axis=2)
    wsum = jnp.sum(anch_n * val_c[:, :, None], axis=1)
    cnt = jnp.maximum(jnp.sum(val_c, axis=1, keepdims=True), 1.0)
    new_center = _normalize(wsum / cnt, axis=1)
    center = _normalize(MU * cluster_center + (1.0 - MU) * new_center, axis=1)
    return (loss, new_queue, center, new_ptr)
